# Initial kernel scaffold; baseline (speedup 1.0000x reference)
import jax
import jax.numpy as jnp
from jax.experimental import pallas as pl


def kernel(ref_embs, p_targets, d_targets, e_targets, params):
    raise NotImplementedError("write your pallas kernel here")


# speedup vs baseline: 1.2475x; 1.2475x over previous
import jax
import jax.numpy as jnp
from jax.experimental import pallas as pl
from jax.experimental.pallas import tpu as pltpu

CHANS = (32, 32, 64, 64, 128, 128)
B = 16
L0 = 2048
T = 32
H = 128


def _enc_kernel(x_ref, w0_ref, b0_ref,
                w1_ref, b1_ref, w2_ref, b2_ref, w3_ref, b3_ref,
                w4_ref, b4_ref, w5_ref, b5_ref,
                wih_ref, whh_ref, bih_ref, bhh_ref,
                e1_ref, e2_ref,
                q_ref, loss_ref, gi_scr):
    f32 = jnp.float32
    x = x_ref[0]
    xr = x.reshape(L0 // 2, 2, B)
    ev = xr[:, 0, :]
    od = xr[:, 1, :]
    pod = jnp.concatenate([jnp.zeros((1, B), f32), od[:-1]], axis=0)
    w0 = w0_ref[0]
    b0 = b0_ref[0]
    h = (pod[:, :, None] * w0[0][None, None, :]
         + ev[:, :, None] * w0[1][None, None, :]
         + od[:, :, None] * w0[2][None, None, :]
         + b0[None, :, :])
    h = jnp.maximum(h, 0.0)

    def conv_layer(h, w_ref, b_ref, c_out):
        Lh, _, c_in = h.shape
        hr = h.reshape(Lh // 2, 2, B, c_in)
        ev = hr[:, 0]
        od = hr[:, 1]
        pod = jnp.concatenate(
            [jnp.zeros((1, B, c_in), f32), od[:-1]], axis=0)
        w = w_ref[0]
        b = b_ref[0]

        def mm(a, wk):
            return (a.reshape(-1, c_in) @ wk).reshape(Lh // 2, B, c_out)

        out = mm(pod, w[0]) + mm(ev, w[1]) + mm(od, w[2]) + b[None, :, :]
        return jnp.maximum(out, 0.0)

    h = conv_layer(h, w1_ref, b1_ref, CHANS[1])
    h = conv_layer(h, w2_ref, b2_ref, CHANS[2])
    h = conv_layer(h, w3_ref, b3_ref, CHANS[3])
    h = conv_layer(h, w4_ref, b4_ref, CHANS[4])
    h = conv_layer(h, w5_ref, b5_ref, CHANS[5])

    wih = wih_ref[0]
    bih = bih_ref[0]
    gi = h.reshape(T * B, H) @ wih + bih
    gi_scr[...] = gi.reshape(T, B, 3 * H)
    whh = whh_ref[0]
    bhh = bhh_ref[0]

    def step(t, hprev):
        git = gi_scr[t]
        gh = hprev @ whh + bhh
        r = jax.nn.sigmoid(git[:, :H] + gh[:, :H])
        z = jax.nn.sigmoid(git[:, H:2 * H] + gh[:, H:2 * H])
        n = jnp.tanh(git[:, 2 * H:] + r * gh[:, 2 * H:])
        return (1.0 - z) * n + z * hprev

    hT = jax.lax.fori_loop(0, T, step, jnp.zeros((B, H), f32))

    def vq_stage(z, emb):
        d = (jnp.sum(z * z, axis=1, keepdims=True)
             - 2.0 * (z @ emb.T)
             + jnp.sum(emb * emb, axis=1)[None, :])
        dmin = jnp.min(d, axis=1, keepdims=True)
        iota = jax.lax.broadcasted_iota(jnp.int32, (B, 7), 1)
        idx = jnp.min(jnp.where(d == dmin, iota, 7), axis=1)
        oh = (idx[:, None] == iota).astype(f32)
        zq = oh @ emb
        e_mean = jnp.mean(oh, axis=0)
        usage = -jnp.sum(e_mean * jnp.log(e_mean + 1e-10))
        loss = 1.4 * jnp.mean((zq - z) ** 2) + 0.01 * usage
        q_st = z + (zq - z)
        return q_st, loss

    q1, l1 = vq_stage(hT, e1_ref[0])
    q2, l2 = vq_stage(hT - q1, e2_ref[0])

    q_ref[...] = jnp.concatenate([q1, q2], axis=1)
    loss_ref[0, 0, :] = jnp.full((H,), l1 + l2, f32)


def _att_kernel(q_ref, re_ref, loss_ref, dp_ref, out_ref, l_ref):
    x = q_ref[...]
    p = dp_ref[...]
    nB, nF = x.shape
    f32 = jnp.float32

    def lrelu(a):
        return jnp.where(a >= 0, a, 0.01 * a)

    def tap3(a, d, k):
        left = jnp.concatenate(
            [jnp.zeros((nB, d), f32), a[:, :-d]], axis=1)
        right = jnp.concatenate(
            [a[:, d:], jnp.zeros((nB, d), f32)], axis=1)
        return p[0, k] * left + p[0, k + 1] * a + p[0, k + 2] * right + p[0, k + 3]

    h = lrelu(tap3(x, 1, 0))
    h = lrelu(tap3(h, 3, 4))
    fp = tap3(h, 5, 8) + x
    gap = jnp.mean(fp, axis=1, keepdims=True)
    gmp = jnp.max(fp, axis=1, keepdims=True)
    c1 = lrelu(p[0, 12] * gap + p[0, 13] * gmp)
    wc = jax.nn.sigmoid(p[0, 14] * c1)
    wt = jax.nn.sigmoid(p[0, 15])
    out_ref[...] = re_ref[...] + fp * (wc * wt)
    l_ref[0, :] = jnp.sum(loss_ref[...][:, 0, :], axis=0)


def _fold_conv(enc, i):
    w = enc['conv%d_w' % i]
    s = enc['bn%d_g' % i] / jnp.sqrt(enc['bn%d_v' % i] + 1e-5)
    bias = enc['bn%d_b' % i] - enc['bn%d_m' % i] * s
    ws = w * s[:, None, None]
    wt = jnp.transpose(ws, (2, 1, 0))
    return wt, bias[None, :]


def kernel(ref_embs, p_targets, d_targets, e_targets, params):
    f32 = jnp.float32
    encs = [params['enc_p'], params['enc_d'], params['enc_e']]

    xs = jnp.stack([p_targets, d_targets, e_targets], axis=0)
    xs = jnp.transpose(xs, (0, 2, 1))

    ws, bs = [], []
    for i in range(6):
        wi, bi = [], []
        for enc in encs:
            w, b = _fold_conv(enc, i)
            wi.append(w)
            bi.append(b)
        ws.append(jnp.stack(wi, 0))
        bs.append(jnp.stack(bi, 0))
    w0 = ws[0][:, :, 0, :]

    wih = jnp.stack([e['W_ih'].T for e in encs], 0)
    whh = jnp.stack([e['W_hh'].T for e in encs], 0)
    bih = jnp.stack([e['b_ih'][None, :] for e in encs], 0)
    bhh = jnp.stack([e['b_hh'][None, :] for e in encs], 0)
    e1 = jnp.stack([params['vq_p_1'], params['vq_d_1'], params['vq_e_1']], 0)
    e2 = jnp.stack([params['vq_p_2'], params['vq_d_2'], params['vq_e_2']], 0)

    def im3(e):
        return (e, 0, 0)

    def im2(e):
        return (0, e)

    q, losses = pl.pallas_call(
        _enc_kernel,
        grid=(3,),
        in_specs=[
            pl.BlockSpec((1, L0, B), im3),
            pl.BlockSpec((1, 3, CHANS[0]), im3),
            pl.BlockSpec((1, 1, CHANS[0]), im3),
            pl.BlockSpec((1, 3, CHANS[0], CHANS[1]), lambda e: (e, 0, 0, 0)),
            pl.BlockSpec((1, 1, CHANS[1]), im3),
            pl.BlockSpec((1, 3, CHANS[1], CHANS[2]), lambda e: (e, 0, 0, 0)),
            pl.BlockSpec((1, 1, CHANS[2]), im3),
            pl.BlockSpec((1, 3, CHANS[2], CHANS[3]), lambda e: (e, 0, 0, 0)),
            pl.BlockSpec((1, 1, CHANS[3]), im3),
            pl.BlockSpec((1, 3, CHANS[3], CHANS[4]), lambda e: (e, 0, 0, 0)),
            pl.BlockSpec((1, 1, CHANS[4]), im3),
            pl.BlockSpec((1, 3, CHANS[4], CHANS[5]), lambda e: (e, 0, 0, 0)),
            pl.BlockSpec((1, 1, CHANS[5]), im3),
            pl.BlockSpec((1, H, 3 * H), im3),
            pl.BlockSpec((1, H, 3 * H), im3),
            pl.BlockSpec((1, 1, 3 * H), im3),
            pl.BlockSpec((1, 1, 3 * H), im3),
            pl.BlockSpec((1, 7, H), im3),
            pl.BlockSpec((1, 7, H), im3),
        ],
        out_specs=[
            pl.BlockSpec((B, 2 * H), im2),
            pl.BlockSpec((1, 1, H), im3),
        ],
        out_shape=[
            jax.ShapeDtypeStruct((B, 6 * H), f32),
            jax.ShapeDtypeStruct((3, 1, H), f32),
        ],
        scratch_shapes=[pltpu.VMEM((T, B, 3 * H), f32)],
    )(xs, w0, bs[0], ws[1], bs[1], ws[2], bs[2], ws[3], bs[3],
      ws[4], bs[4], ws[5], bs[5], wih, whh, bih, bhh, e1, e2)

    da = params['da']
    dp = jnp.concatenate([
        da['rb_w1'][0, 0], da['rb_b1'],
        da['rb_w2'][0, 0], da['rb_b2'],
        da['rb_w3'][0, 0], da['rb_b3'],
        da['ca_w1'][0, :, 0], da['ca_w2'][0, 0], da['ta_b'],
    ]).reshape(1, 16).astype(f32)

    out, ltot = pl.pallas_call(
        _att_kernel,
        in_specs=[
            pl.BlockSpec((B, 6 * H), lambda: (0, 0)),
            pl.BlockSpec((B, 6 * H), lambda: (0, 0)),
            pl.BlockSpec((3, 1, H), lambda: (0, 0, 0)),
            pl.BlockSpec((1, 16), lambda: (0, 0)),
        ],
        out_specs=[
            pl.BlockSpec((B, 6 * H), lambda: (0, 0)),
            pl.BlockSpec((1, H), lambda: (0, 0)),
        ],
        out_shape=[
            jax.ShapeDtypeStruct((B, 6 * H), f32),
            jax.ShapeDtypeStruct((1, H), f32),
        ],
    )(q, ref_embs, losses, dp)

    return out, ltot[0, 0]


# speedup vs baseline: 1.2495x; 1.0016x over previous
import jax
import jax.numpy as jnp
from jax.experimental import pallas as pl
from jax.experimental.pallas import tpu as pltpu

CHANS = (32, 32, 64, 64, 128, 128)
B = 16
L0 = 2048
T = 32
H = 128


def _enc_kernel(x_ref, w0_ref, b0_ref,
                w1_ref, b1_ref, w2_ref, b2_ref, w3_ref, b3_ref,
                w4_ref, b4_ref, w5_ref, b5_ref,
                wih_ref, whh_ref, bih_ref, bhh_ref,
                e1_ref, e2_ref,
                q_ref, loss_ref, gi_scr):
    f32 = jnp.float32
    x = x_ref[0]
    xr = x.reshape(L0 // 2, 2, B)
    ev = xr[:, 0, :]
    od = xr[:, 1, :]
    pod = jnp.concatenate([jnp.zeros((1, B), f32), od[:-1]], axis=0)
    w0 = w0_ref[0]
    b0 = b0_ref[0]
    h = (pod[:, :, None] * w0[0][None, None, :]
         + ev[:, :, None] * w0[1][None, None, :]
         + od[:, :, None] * w0[2][None, None, :]
         + b0[None, :, :])
    h = jnp.maximum(h, 0.0)

    def conv_layer(h, w_ref, b_ref, c_out):
        Lh, _, c_in = h.shape
        hr = h.reshape(Lh // 2, 2, B, c_in)
        ev = hr[:, 0]
        od = hr[:, 1]
        pod = jnp.concatenate(
            [jnp.zeros((1, B, c_in), f32), od[:-1]], axis=0)
        w = w_ref[0]
        b = b_ref[0]

        def mm(a, wk):
            return (a.reshape(-1, c_in) @ wk).reshape(Lh // 2, B, c_out)

        out = mm(pod, w[0]) + mm(ev, w[1]) + mm(od, w[2]) + b[None, :, :]
        return jnp.maximum(out, 0.0)

    h = conv_layer(h, w1_ref, b1_ref, CHANS[1])
    h = conv_layer(h, w2_ref, b2_ref, CHANS[2])
    h = conv_layer(h, w3_ref, b3_ref, CHANS[3])
    h = conv_layer(h, w4_ref, b4_ref, CHANS[4])
    h = conv_layer(h, w5_ref, b5_ref, CHANS[5])

    wih = wih_ref[0]
    bih = bih_ref[0]
    gi = h.reshape(T * B, H) @ wih + bih
    gi_scr[...] = gi.reshape(T, B, 3 * H)
    whh = whh_ref[0]
    bhh = bhh_ref[0]

    def step(t, hprev):
        git = gi_scr[t]
        gh = hprev @ whh + bhh
        r = jax.nn.sigmoid(git[:, :H] + gh[:, :H])
        z = jax.nn.sigmoid(git[:, H:2 * H] + gh[:, H:2 * H])
        n = jnp.tanh(git[:, 2 * H:] + r * gh[:, 2 * H:])
        return (1.0 - z) * n + z * hprev

    hT = jax.lax.fori_loop(0, T, step, jnp.zeros((B, H), f32))

    def vq_stage(z, emb):
        d = (jnp.sum(z * z, axis=1, keepdims=True)
             - 2.0 * (z @ emb.T)
             + jnp.sum(emb * emb, axis=1)[None, :])
        dmin = jnp.min(d, axis=1, keepdims=True)
        iota = jax.lax.broadcasted_iota(jnp.int32, (B, 7), 1)
        idx = jnp.min(jnp.where(d == dmin, iota, 7), axis=1)
        oh = (idx[:, None] == iota).astype(f32)
        zq = oh @ emb
        e_mean = jnp.mean(oh, axis=0)
        usage = -jnp.sum(e_mean * jnp.log(e_mean + 1e-10))
        loss = 1.4 * jnp.mean((zq - z) ** 2) + 0.01 * usage
        q_st = z + (zq - z)
        return q_st, loss

    q1, l1 = vq_stage(hT, e1_ref[0])
    q2, l2 = vq_stage(hT - q1, e2_ref[0])

    q_ref[...] = jnp.concatenate([q1, q2], axis=1)
    loss_ref[0, 0, :] = jnp.full((H,), l1 + l2, f32)


def _att_kernel(q_ref, re_ref, loss_ref, dp_ref, out_ref, l_ref):
    x = q_ref[...]
    p = dp_ref[...]
    nB, nF = x.shape
    f32 = jnp.float32

    def lrelu(a):
        return jnp.where(a >= 0, a, 0.01 * a)

    def tap3(a, d, k):
        left = jnp.concatenate(
            [jnp.zeros((nB, d), f32), a[:, :-d]], axis=1)
        right = jnp.concatenate(
            [a[:, d:], jnp.zeros((nB, d), f32)], axis=1)
        return p[0, k] * left + p[0, k + 1] * a + p[0, k + 2] * right + p[0, k + 3]

    h = lrelu(tap3(x, 1, 0))
    h = lrelu(tap3(h, 3, 4))
    fp = tap3(h, 5, 8) + x
    gap = jnp.mean(fp, axis=1, keepdims=True)
    gmp = jnp.max(fp, axis=1, keepdims=True)
    c1 = lrelu(p[0, 12] * gap + p[0, 13] * gmp)
    wc = jax.nn.sigmoid(p[0, 14] * c1)
    wt = jax.nn.sigmoid(p[0, 15])
    out_ref[...] = re_ref[...] + fp * (wc * wt)
    l_ref[0, :] = jnp.sum(loss_ref[...][:, 0, :], axis=0)


def _fold_conv(enc, i):
    w = enc['conv%d_w' % i]
    s = enc['bn%d_g' % i] / jnp.sqrt(enc['bn%d_v' % i] + 1e-5)
    bias = enc['bn%d_b' % i] - enc['bn%d_m' % i] * s
    ws = w * s[:, None, None]
    wt = jnp.transpose(ws, (2, 1, 0))
    return wt, bias[None, :]


def kernel(ref_embs, p_targets, d_targets, e_targets, params):
    f32 = jnp.float32
    encs = [params['enc_p'], params['enc_d'], params['enc_e']]

    xs = jnp.stack([p_targets, d_targets, e_targets], axis=0)
    xs = jnp.transpose(xs, (0, 2, 1))

    ws, bs = [], []
    for i in range(6):
        wi, bi = [], []
        for enc in encs:
            w, b = _fold_conv(enc, i)
            wi.append(w)
            bi.append(b)
        ws.append(jnp.stack(wi, 0))
        bs.append(jnp.stack(bi, 0))
    w0 = ws[0][:, :, 0, :]

    wih = jnp.stack([e['W_ih'].T for e in encs], 0)
    whh = jnp.stack([e['W_hh'].T for e in encs], 0)
    bih = jnp.stack([e['b_ih'][None, :] for e in encs], 0)
    bhh = jnp.stack([e['b_hh'][None, :] for e in encs], 0)
    e1 = jnp.stack([params['vq_p_1'], params['vq_d_1'], params['vq_e_1']], 0)
    e2 = jnp.stack([params['vq_p_2'], params['vq_d_2'], params['vq_e_2']], 0)

    def im3(e):
        return (e, 0, 0)

    def im2(e):
        return (0, e)

    q, losses = pl.pallas_call(
        _enc_kernel,
        grid=(3,),
        in_specs=[
            pl.BlockSpec((1, L0, B), im3),
            pl.BlockSpec((1, 3, CHANS[0]), im3),
            pl.BlockSpec((1, 1, CHANS[0]), im3),
            pl.BlockSpec((1, 3, CHANS[0], CHANS[1]), lambda e: (e, 0, 0, 0)),
            pl.BlockSpec((1, 1, CHANS[1]), im3),
            pl.BlockSpec((1, 3, CHANS[1], CHANS[2]), lambda e: (e, 0, 0, 0)),
            pl.BlockSpec((1, 1, CHANS[2]), im3),
            pl.BlockSpec((1, 3, CHANS[2], CHANS[3]), lambda e: (e, 0, 0, 0)),
            pl.BlockSpec((1, 1, CHANS[3]), im3),
            pl.BlockSpec((1, 3, CHANS[3], CHANS[4]), lambda e: (e, 0, 0, 0)),
            pl.BlockSpec((1, 1, CHANS[4]), im3),
            pl.BlockSpec((1, 3, CHANS[4], CHANS[5]), lambda e: (e, 0, 0, 0)),
            pl.BlockSpec((1, 1, CHANS[5]), im3),
            pl.BlockSpec((1, H, 3 * H), im3),
            pl.BlockSpec((1, H, 3 * H), im3),
            pl.BlockSpec((1, 1, 3 * H), im3),
            pl.BlockSpec((1, 1, 3 * H), im3),
            pl.BlockSpec((1, 7, H), im3),
            pl.BlockSpec((1, 7, H), im3),
        ],
        out_specs=[
            pl.BlockSpec((B, 2 * H), im2),
            pl.BlockSpec((1, 1, H), im3),
        ],
        out_shape=[
            jax.ShapeDtypeStruct((B, 6 * H), f32),
            jax.ShapeDtypeStruct((3, 1, H), f32),
        ],
        scratch_shapes=[pltpu.VMEM((T, B, 3 * H), f32)],
        compiler_params=pltpu.CompilerParams(
            dimension_semantics=("parallel",)),
    )(xs, w0, bs[0], ws[1], bs[1], ws[2], bs[2], ws[3], bs[3],
      ws[4], bs[4], ws[5], bs[5], wih, whh, bih, bhh, e1, e2)

    da = params['da']
    dp = jnp.concatenate([
        da['rb_w1'][0, 0], da['rb_b1'],
        da['rb_w2'][0, 0], da['rb_b2'],
        da['rb_w3'][0, 0], da['rb_b3'],
        da['ca_w1'][0, :, 0], da['ca_w2'][0, 0], da['ta_b'],
    ]).reshape(1, 16).astype(f32)

    out, ltot = pl.pallas_call(
        _att_kernel,
        in_specs=[
            pl.BlockSpec((B, 6 * H), lambda: (0, 0)),
            pl.BlockSpec((B, 6 * H), lambda: (0, 0)),
            pl.BlockSpec((3, 1, H), lambda: (0, 0, 0)),
            pl.BlockSpec((1, 16), lambda: (0, 0)),
        ],
        out_specs=[
            pl.BlockSpec((B, 6 * H), lambda: (0, 0)),
            pl.BlockSpec((1, H), lambda: (0, 0)),
        ],
        out_shape=[
            jax.ShapeDtypeStruct((B, 6 * H), f32),
            jax.ShapeDtypeStruct((1, H), f32),
        ],
    )(q, ref_embs, losses, dp)

    return out, ltot[0, 0]


# speedup vs baseline: 1.4314x; 1.1456x over previous
import jax
import jax.numpy as jnp
from jax.experimental import pallas as pl
from jax.experimental.pallas import tpu as pltpu

CHANS = (32, 32, 64, 64, 128, 128)
B = 16
L0 = 2048
T = 32
H = 128
NE = 3
NB = NE * B
NC = 7


def _taps(h):
    Lh = h.shape[0]
    hr = h.reshape(Lh // 2, 2, *h.shape[1:])
    ev = hr[:, 0]
    od = hr[:, 1]
    pod = jnp.concatenate(
        [jnp.zeros((1,) + od.shape[1:], jnp.float32), od[:-1]], axis=0)
    return pod, ev, od


def _fused_kernel(x_ref, w0_ref, b0_ref,
                  w1_ref, b1_ref, w2_ref, b2_ref, w3_ref, b3_ref,
                  w4_ref, b4_ref, w5_ref, b5_ref,
                  wih_ref, whh_ref, bih_ref, bhh_ref,
                  e1_ref, e2_ref, re_ref, dp_ref,
                  out_ref, l_ref, gi_scr, q_scr):
    f32 = jnp.float32
    w_refs = (w1_ref, w2_ref, w3_ref, w4_ref, w5_ref)
    b_refs = (b1_ref, b2_ref, b3_ref, b4_ref, b5_ref)

    for e in range(NE):
        x = x_ref[e]
        pod, ev, od = _taps(x)
        w0 = w0_ref[e]
        h = (pod[:, :, None] * w0[0][None, None, :]
             + ev[:, :, None] * w0[1][None, None, :]
             + od[:, :, None] * w0[2][None, None, :]
             + b0_ref[e][None, :, :])
        h = jnp.maximum(h, 0.0)

        for l in range(5):
            c_in, c_out = CHANS[l], CHANS[l + 1]
            pod, ev, od = _taps(h)
            cat = jnp.concatenate([pod, ev, od], axis=2)
            rows = cat.shape[0] * B
            out = cat.reshape(rows, 3 * c_in) @ w_refs[l][e] + b_refs[l][e]
            h = jnp.maximum(out, 0.0).reshape(cat.shape[0], B, c_out)

        gi = h.reshape(T * B, H) @ wih_ref[e] + bih_ref[e]
        gi_scr[:, B * e:B * (e + 1), :] = gi.reshape(T, B, 3 * H)

    def step(t, hall):
        git = gi_scr[t]
        gh = jnp.concatenate(
            [hall[B * e:B * (e + 1)] @ whh_ref[e] + bhh_ref[e]
             for e in range(NE)], axis=0)
        r = jax.nn.sigmoid(git[:, :H] + gh[:, :H])
        z = jax.nn.sigmoid(git[:, H:2 * H] + gh[:, H:2 * H])
        n = jnp.tanh(git[:, 2 * H:] + r * gh[:, 2 * H:])
        return (1.0 - z) * n + z * hall

    hT = jax.lax.fori_loop(0, T, step, jnp.zeros((NB, H), f32))

    rg = jax.lax.broadcasted_iota(jnp.int32, (NB, NE * NC), 0) // B
    cg = jax.lax.broadcasted_iota(jnp.int32, (NB, NE * NC), 1)

    def vq_batch(z, ecat):
        d = (jnp.sum(z * z, axis=1, keepdims=True)
             - 2.0 * (z @ ecat.T)
             + jnp.sum(ecat * ecat, axis=1)[None, :])
        d = jnp.where(rg == (cg // NC), d, 1e30)
        dmin = jnp.min(d, axis=1, keepdims=True)
        idx = jnp.min(jnp.where(d == dmin, cg, NE * NC), axis=1)
        oh = (idx[:, None] == cg).astype(f32)
        zq = oh @ ecat
        seg = jnp.mean(oh.reshape(NE, B, NE * NC), axis=1)
        usage = -jnp.sum(seg * jnp.log(seg + 1e-10))
        loss = 1.4 * jnp.sum((zq - z) ** 2) / (B * H) + 0.01 * usage
        return z + (zq - z), loss

    ecat1 = jnp.concatenate([e1_ref[e] for e in range(NE)], axis=0)
    ecat2 = jnp.concatenate([e2_ref[e] for e in range(NE)], axis=0)
    q1, l1 = vq_batch(hT, ecat1)
    q2, l2 = vq_batch(hT - q1, ecat2)
    qa = jnp.concatenate([q1, q2], axis=1)
    for e in range(NE):
        q_scr[:, 2 * H * e:2 * H * (e + 1)] = qa[B * e:B * (e + 1)]

    x = q_scr[...]
    p = dp_ref[...]

    def lrelu(a):
        return jnp.where(a >= 0, a, 0.01 * a)

    def tap3(a, d, k):
        left = jnp.concatenate(
            [jnp.zeros((B, d), f32), a[:, :-d]], axis=1)
        right = jnp.concatenate(
            [a[:, d:], jnp.zeros((B, d), f32)], axis=1)
        return (p[0, k] * left + p[0, k + 1] * a
                + p[0, k + 2] * right + p[0, k + 3])

    hh = lrelu(tap3(x, 1, 0))
    hh = lrelu(tap3(hh, 3, 4))
    fp = tap3(hh, 5, 8) + x
    gap = jnp.mean(fp, axis=1, keepdims=True)
    gmp = jnp.max(fp, axis=1, keepdims=True)
    c1 = lrelu(p[0, 12] * gap + p[0, 13] * gmp)
    wc = jax.nn.sigmoid(p[0, 14] * c1)
    wt = jax.nn.sigmoid(p[0, 15])
    out_ref[...] = re_ref[...] + fp * (wc * wt)
    l_ref[0, :] = jnp.full((H,), l1 + l2, f32)


def _fold_conv(enc, i):
    w = enc['conv%d_w' % i]
    s = enc['bn%d_g' % i] / jnp.sqrt(enc['bn%d_v' % i] + 1e-5)
    bias = enc['bn%d_b' % i] - enc['bn%d_m' % i] * s
    ws = w * s[:, None, None]
    wt = jnp.transpose(ws, (2, 1, 0))
    return wt, bias[None, :]


def kernel(ref_embs, p_targets, d_targets, e_targets, params):
    f32 = jnp.float32
    encs = [params['enc_p'], params['enc_d'], params['enc_e']]

    xs = jnp.stack([p_targets, d_targets, e_targets], axis=0)
    xs = jnp.transpose(xs, (0, 2, 1))

    ws, bs = [], []
    for i in range(6):
        wi, bi = [], []
        for enc in encs:
            w, b = _fold_conv(enc, i)
            wi.append(w)
            bi.append(b)
        ic = 1 if i == 0 else CHANS[i - 1]
        ws.append(jnp.stack(wi, 0).reshape(NE, 3 * ic, CHANS[i]))
        bs.append(jnp.stack(bi, 0))
    w0 = ws[0]

    wih = jnp.stack([e['W_ih'].T for e in encs], 0)
    whh = jnp.stack([e['W_hh'].T for e in encs], 0)
    bih = jnp.stack([e['b_ih'][None, :] for e in encs], 0)
    bhh = jnp.stack([e['b_hh'][None, :] for e in encs], 0)
    e1 = jnp.stack([params['vq_p_1'], params['vq_d_1'], params['vq_e_1']], 0)
    e2 = jnp.stack([params['vq_p_2'], params['vq_d_2'], params['vq_e_2']], 0)

    da = params['da']
    dp = jnp.concatenate([
        da['rb_w1'][0, 0], da['rb_b1'],
        da['rb_w2'][0, 0], da['rb_b2'],
        da['rb_w3'][0, 0], da['rb_b3'],
        da['ca_w1'][0, :, 0], da['ca_w2'][0, 0], da['ta_b'],
    ]).reshape(1, 16).astype(f32)

    out, ltot = pl.pallas_call(
        _fused_kernel,
        out_shape=[
            jax.ShapeDtypeStruct((B, 6 * H), f32),
            jax.ShapeDtypeStruct((1, H), f32),
        ],
        scratch_shapes=[
            pltpu.VMEM((T, NB, 3 * H), f32),
            pltpu.VMEM((B, 6 * H), f32),
        ],
    )(xs, w0, bs[0], ws[1], bs[1], ws[2], bs[2], ws[3], bs[3],
      ws[4], bs[4], ws[5], bs[5], wih, whh, bih, bhh, e1, e2,
      ref_embs, dp)

    return out, ltot[0, 0]


# speedup vs baseline: 1.9304x; 1.3486x over previous
import jax
import jax.numpy as jnp
from jax.experimental import pallas as pl
from jax.experimental.pallas import tpu as pltpu

CHANS = (32, 32, 64, 64, 128, 128)
B = 16
L0 = 2048
T = 32
H = 128
NE = 3
NB = NE * B
NC = 7


def _fused_kernel(x_ref, w0_ref, b0_ref, wa_ref, wc_ref, bc_ref,
                  wih_ref, whh_ref, bih_ref, bhh_ref,
                  e1_ref, e2_ref, re_ref, dp_ref,
                  out_ref, l_ref, gi_scr, q_scr):
    f32 = jnp.float32

    for e in range(NE):
        x = x_ref[e]
        xr = x.reshape(L0 // 2, 2, B)
        ev = xr[:, 0, :]
        od = xr[:, 1, :]
        pod = jnp.concatenate(
            [jnp.zeros((1, B), f32), od[:-1]], axis=0)
        w0 = w0_ref[e]
        h = (pod[:, :, None] * w0[0][None, None, :]
             + ev[:, :, None] * w0[1][None, None, :]
             + od[:, :, None] * w0[2][None, None, :]
             + b0_ref[e][None, :, :])
        h = jnp.maximum(h, 0.0).reshape(L0 // 2 * B, H)

        for l in range(5):
            rows = h.shape[0]
            hr = h.reshape(rows // (2 * B), 2 * B, H)
            evf = hr[:, :B, :].reshape(rows // 2, H)
            odf = hr[:, B:, :].reshape(rows // 2, H)
            a = evf @ wa_ref[l, e] + bc_ref[l, e]
            c = (odf @ wc_ref[l, e]).reshape(rows // (2 * B), B, 2 * H)
            c0 = c[:, :, :H]
            c2 = c[:, :, H:]
            c0s = jnp.concatenate(
                [jnp.zeros((1, B, H), f32), c0[:-1]], axis=0)
            h = jnp.maximum(
                a.reshape(rows // (2 * B), B, H) + c2 + c0s, 0.0)
            h = h.reshape(rows // 2, H)

        gi = h @ wih_ref[e] + bih_ref[e]
        gi_scr[:, B * e:B * (e + 1), :] = gi.reshape(T, B, 3 * H)

    def step(t, hall):
        git = gi_scr[t]
        gh = jnp.concatenate(
            [hall[B * e:B * (e + 1)] @ whh_ref[e] + bhh_ref[e]
             for e in range(NE)], axis=0)
        r = jax.nn.sigmoid(git[:, :H] + gh[:, :H])
        z = jax.nn.sigmoid(git[:, H:2 * H] + gh[:, H:2 * H])
        n = jnp.tanh(git[:, 2 * H:] + r * gh[:, 2 * H:])
        return (1.0 - z) * n + z * hall

    hT = jax.lax.fori_loop(0, T, step, jnp.zeros((NB, H), f32))

    rg = jax.lax.broadcasted_iota(jnp.int32, (NB, NE * NC), 0) // B
    cg = jax.lax.broadcasted_iota(jnp.int32, (NB, NE * NC), 1)

    def vq_batch(z, ecat):
        d = (jnp.sum(z * z, axis=1, keepdims=True)
             - 2.0 * (z @ ecat.T)
             + jnp.sum(ecat * ecat, axis=1)[None, :])
        d = jnp.where(rg == (cg // NC), d, 1e30)
        dmin = jnp.min(d, axis=1, keepdims=True)
        idx = jnp.min(jnp.where(d == dmin, cg, NE * NC), axis=1)
        oh = (idx[:, None] == cg).astype(f32)
        zq = oh @ ecat
        seg = jnp.mean(oh.reshape(NE, B, NE * NC), axis=1)
        usage = -jnp.sum(seg * jnp.log(seg + 1e-10))
        loss = 1.4 * jnp.sum((zq - z) ** 2) / (B * H) + 0.01 * usage
        return z + (zq - z), loss

    ecat1 = jnp.concatenate([e1_ref[e] for e in range(NE)], axis=0)
    ecat2 = jnp.concatenate([e2_ref[e] for e in range(NE)], axis=0)
    q1, l1 = vq_batch(hT, ecat1)
    q2, l2 = vq_batch(hT - q1, ecat2)
    qa = jnp.concatenate([q1, q2], axis=1)
    for e in range(NE):
        q_scr[:, 2 * H * e:2 * H * (e + 1)] = qa[B * e:B * (e + 1)]

    x = q_scr[...]
    p = dp_ref[...]

    def lrelu(a):
        return jnp.where(a >= 0, a, 0.01 * a)

    def tap3(a, d, k):
        left = jnp.concatenate(
            [jnp.zeros((B, d), f32), a[:, :-d]], axis=1)
        right = jnp.concatenate(
            [a[:, d:], jnp.zeros((B, d), f32)], axis=1)
        return (p[0, k] * left + p[0, k + 1] * a
                + p[0, k + 2] * right + p[0, k + 3])

    hh = lrelu(tap3(x, 1, 0))
    hh = lrelu(tap3(hh, 3, 4))
    fp = tap3(hh, 5, 8) + x
    gap = jnp.mean(fp, axis=1, keepdims=True)
    gmp = jnp.max(fp, axis=1, keepdims=True)
    c1 = lrelu(p[0, 12] * gap + p[0, 13] * gmp)
    wc = jax.nn.sigmoid(p[0, 14] * c1)
    wt = jax.nn.sigmoid(p[0, 15])
    out_ref[...] = re_ref[...] + fp * (wc * wt)
    l_ref[0, :] = jnp.full((H,), l1 + l2, f32)


def _fold_conv(enc, i):
    w = enc['conv%d_w' % i]
    s = enc['bn%d_g' % i] / jnp.sqrt(enc['bn%d_v' % i] + 1e-5)
    bias = enc['bn%d_b' % i] - enc['bn%d_m' % i] * s
    ws = w * s[:, None, None]
    wt = jnp.transpose(ws, (2, 1, 0))
    return wt, bias[None, :]


def kernel(ref_embs, p_targets, d_targets, e_targets, params):
    f32 = jnp.float32
    encs = [params['enc_p'], params['enc_d'], params['enc_e']]

    xs = jnp.stack([p_targets, d_targets, e_targets], axis=0)
    xs = jnp.transpose(xs, (0, 2, 1))

    wa_l, wc_l, bc_l = [], [], []
    w0_l, b0_l = [], []
    for enc in encs:
        w, b = _fold_conv(enc, 0)
        w0_l.append(jnp.pad(w[:, 0, :], ((0, 0), (0, H - CHANS[0]))))
        b0_l.append(jnp.pad(b, ((0, 0), (0, H - CHANS[0]))))
    w0 = jnp.stack(w0_l, 0)
    b0 = jnp.stack(b0_l, 0)
    for i in range(1, 6):
        ic, oc = CHANS[i - 1], CHANS[i]
        wa_e, wc_e, bc_e = [], [], []
        for enc in encs:
            w, b = _fold_conv(enc, i)
            wa_e.append(jnp.pad(w[1], ((0, H - ic), (0, H - oc))))
            wc_e.append(jnp.pad(
                jnp.concatenate(
                    [jnp.pad(w[0], ((0, 0), (0, H - oc))),
                     jnp.pad(w[2], ((0, 0), (0, H - oc)))], axis=1),
                ((0, H - ic), (0, 0))))
            bc_e.append(jnp.pad(b, ((0, 0), (0, H - oc))))
        wa_l.append(jnp.stack(wa_e, 0))
        wc_l.append(jnp.stack(wc_e, 0))
        bc_l.append(jnp.stack(bc_e, 0))
    wa = jnp.stack(wa_l, 0)
    wc = jnp.stack(wc_l, 0)
    bc = jnp.stack(bc_l, 0)

    wih = jnp.stack([e['W_ih'].T for e in encs], 0)
    whh = jnp.stack([e['W_hh'].T for e in encs], 0)
    bih = jnp.stack([e['b_ih'][None, :] for e in encs], 0)
    bhh = jnp.stack([e['b_hh'][None, :] for e in encs], 0)
    e1 = jnp.stack([params['vq_p_1'], params['vq_d_1'], params['vq_e_1']], 0)
    e2 = jnp.stack([params['vq_p_2'], params['vq_d_2'], params['vq_e_2']], 0)

    da = params['da']
    dp = jnp.concatenate([
        da['rb_w1'][0, 0], da['rb_b1'],
        da['rb_w2'][0, 0], da['rb_b2'],
        da['rb_w3'][0, 0], da['rb_b3'],
        da['ca_w1'][0, :, 0], da['ca_w2'][0, 0], da['ta_b'],
    ]).reshape(1, 16).astype(f32)

    out, ltot = pl.pallas_call(
        _fused_kernel,
        out_shape=[
            jax.ShapeDtypeStruct((B, 6 * H), f32),
            jax.ShapeDtypeStruct((1, H), f32),
        ],
        scratch_shapes=[
            pltpu.VMEM((T, NB, 3 * H), f32),
            pltpu.VMEM((B, 6 * H), f32),
        ],
    )(xs, w0, b0, wa, wc, bc, wih, whh, bih, bhh, e1, e2,
      ref_embs, dp)

    return out, ltot[0, 0]


# speedup vs baseline: 1.9604x; 1.0155x over previous
import jax
import jax.numpy as jnp
from jax.experimental import pallas as pl
from jax.experimental.pallas import tpu as pltpu

CHANS = (32, 32, 64, 64, 128, 128)
B = 16
L0 = 2048
T = 32
H = 128
NE = 3
NB = NE * B
NC = 7


def _fused_kernel(x_ref, w0_ref, b0_ref, wa_ref, wc_ref, bc_ref,
                  wih_ref, whh_ref, bih_ref, bhh_ref,
                  e1_ref, e2_ref, re_ref, dp_ref,
                  out_ref, l_ref, gi_scr, q_scr):
    f32 = jnp.float32

    for e in range(NE):
        x = x_ref[e]
        xr = x.reshape(L0 // 2, 2, B)
        ev = xr[:, 0, :]
        od = xr[:, 1, :]
        pod = jnp.concatenate(
            [jnp.zeros((1, B), f32), od[:-1]], axis=0)
        x48 = jnp.concatenate([pod, ev, od], axis=1)
        h = x48 @ w0_ref[e] + b0_ref[e]
        h = jnp.maximum(h, 0.0).reshape(L0 // 2 * B, H)

        for l in range(5):
            rows = h.shape[0]
            hr = h.reshape(rows // (2 * B), 2 * B, H)
            evf = hr[:, :B, :].reshape(rows // 2, H)
            odf = hr[:, B:, :].reshape(rows // 2, H)
            a = evf @ wa_ref[l, e] + bc_ref[l, e]
            c = (odf @ wc_ref[l, e]).reshape(rows // (2 * B), B, 2 * H)
            c0 = c[:, :, :H]
            c2 = c[:, :, H:]
            c0s = jnp.concatenate(
                [jnp.zeros((1, B, H), f32), c0[:-1]], axis=0)
            h = jnp.maximum(
                a.reshape(rows // (2 * B), B, H) + c2 + c0s, 0.0)
            h = h.reshape(rows // 2, H)

        gi = h @ wih_ref[e] + bih_ref[e]
        gi_scr[:, B * e:B * (e + 1), :] = gi.reshape(T, B, 3 * H)

    def step(t, hall):
        git = gi_scr[t]
        gh = jnp.concatenate(
            [hall[B * e:B * (e + 1)] @ whh_ref[e] + bhh_ref[e]
             for e in range(NE)], axis=0)
        r = jax.nn.sigmoid(git[:, :H] + gh[:, :H])
        z = jax.nn.sigmoid(git[:, H:2 * H] + gh[:, H:2 * H])
        n = jnp.tanh(git[:, 2 * H:] + r * gh[:, 2 * H:])
        return (1.0 - z) * n + z * hall

    hT = jax.lax.fori_loop(0, T, step, jnp.zeros((NB, H), f32), unroll=4)

    rg = jax.lax.broadcasted_iota(jnp.int32, (NB, NE * NC), 0) // B
    cg = jax.lax.broadcasted_iota(jnp.int32, (NB, NE * NC), 1)

    def vq_batch(z, ecat):
        d = (jnp.sum(z * z, axis=1, keepdims=True)
             - 2.0 * (z @ ecat.T)
             + jnp.sum(ecat * ecat, axis=1)[None, :])
        d = jnp.where(rg == (cg // NC), d, 1e30)
        dmin = jnp.min(d, axis=1, keepdims=True)
        idx = jnp.min(jnp.where(d == dmin, cg, NE * NC), axis=1)
        oh = (idx[:, None] == cg).astype(f32)
        zq = oh @ ecat
        seg = jnp.mean(oh.reshape(NE, B, NE * NC), axis=1)
        usage = -jnp.sum(seg * jnp.log(seg + 1e-10))
        loss = 1.4 * jnp.sum((zq - z) ** 2) / (B * H) + 0.01 * usage
        return z + (zq - z), loss

    ecat1 = jnp.concatenate([e1_ref[e] for e in range(NE)], axis=0)
    ecat2 = jnp.concatenate([e2_ref[e] for e in range(NE)], axis=0)
    q1, l1 = vq_batch(hT, ecat1)
    q2, l2 = vq_batch(hT - q1, ecat2)
    qa = jnp.concatenate([q1, q2], axis=1)
    for e in range(NE):
        q_scr[:, 2 * H * e:2 * H * (e + 1)] = qa[B * e:B * (e + 1)]

    x = q_scr[...]
    p = dp_ref[...]

    def lrelu(a):
        return jnp.where(a >= 0, a, 0.01 * a)

    def tap3(a, d, k):
        left = jnp.concatenate(
            [jnp.zeros((B, d), f32), a[:, :-d]], axis=1)
        right = jnp.concatenate(
            [a[:, d:], jnp.zeros((B, d), f32)], axis=1)
        return (p[0, k] * left + p[0, k + 1] * a
                + p[0, k + 2] * right + p[0, k + 3])

    hh = lrelu(tap3(x, 1, 0))
    hh = lrelu(tap3(hh, 3, 4))
    fp = tap3(hh, 5, 8) + x
    gap = jnp.mean(fp, axis=1, keepdims=True)
    gmp = jnp.max(fp, axis=1, keepdims=True)
    c1 = lrelu(p[0, 12] * gap + p[0, 13] * gmp)
    wc = jax.nn.sigmoid(p[0, 14] * c1)
    wt = jax.nn.sigmoid(p[0, 15])
    out_ref[...] = re_ref[...] + fp * (wc * wt)
    l_ref[0, :] = jnp.full((H,), l1 + l2, f32)


def _fold_conv(enc, i):
    w = enc['conv%d_w' % i]
    s = enc['bn%d_g' % i] / jnp.sqrt(enc['bn%d_v' % i] + 1e-5)
    bias = enc['bn%d_b' % i] - enc['bn%d_m' % i] * s
    ws = w * s[:, None, None]
    wt = jnp.transpose(ws, (2, 1, 0))
    return wt, bias[None, :]


def kernel(ref_embs, p_targets, d_targets, e_targets, params):
    f32 = jnp.float32
    encs = [params['enc_p'], params['enc_d'], params['enc_e']]

    xs = jnp.stack([p_targets, d_targets, e_targets], axis=0)
    xs = jnp.transpose(xs, (0, 2, 1))

    wa_l, wc_l, bc_l = [], [], []
    w0_l, b0_l = [], []
    eyeb = jnp.eye(B, dtype=f32)[:, :, None]
    for enc in encs:
        w, b = _fold_conv(enc, 0)
        wp = jnp.pad(w[:, 0, :], ((0, 0), (0, H - CHANS[0])))
        w48 = (eyeb * wp[:, None, None, :]).reshape(3 * B, B * H)
        w0_l.append(w48)
        b0_l.append(jnp.tile(jnp.pad(b, ((0, 0), (0, H - CHANS[0]))),
                             (1, B)))
    w0 = jnp.stack(w0_l, 0)
    b0 = jnp.stack(b0_l, 0)
    for i in range(1, 6):
        ic, oc = CHANS[i - 1], CHANS[i]
        wa_e, wc_e, bc_e = [], [], []
        for enc in encs:
            w, b = _fold_conv(enc, i)
            wa_e.append(jnp.pad(w[1], ((0, H - ic), (0, H - oc))))
            wc_e.append(jnp.pad(
                jnp.concatenate(
                    [jnp.pad(w[0], ((0, 0), (0, H - oc))),
                     jnp.pad(w[2], ((0, 0), (0, H - oc)))], axis=1),
                ((0, H - ic), (0, 0))))
            bc_e.append(jnp.pad(b, ((0, 0), (0, H - oc))))
        wa_l.append(jnp.stack(wa_e, 0))
        wc_l.append(jnp.stack(wc_e, 0))
        bc_l.append(jnp.stack(bc_e, 0))
    wa = jnp.stack(wa_l, 0)
    wc = jnp.stack(wc_l, 0)
    bc = jnp.stack(bc_l, 0)

    wih = jnp.stack([e['W_ih'].T for e in encs], 0)
    whh = jnp.stack([e['W_hh'].T for e in encs], 0)
    bih = jnp.stack([e['b_ih'][None, :] for e in encs], 0)
    bhh = jnp.stack([e['b_hh'][None, :] for e in encs], 0)
    e1 = jnp.stack([params['vq_p_1'], params['vq_d_1'], params['vq_e_1']], 0)
    e2 = jnp.stack([params['vq_p_2'], params['vq_d_2'], params['vq_e_2']], 0)

    da = params['da']
    dp = jnp.concatenate([
        da['rb_w1'][0, 0], da['rb_b1'],
        da['rb_w2'][0, 0], da['rb_b2'],
        da['rb_w3'][0, 0], da['rb_b3'],
        da['ca_w1'][0, :, 0], da['ca_w2'][0, 0], da['ta_b'],
    ]).reshape(1, 16).astype(f32)

    out, ltot = pl.pallas_call(
        _fused_kernel,
        out_shape=[
            jax.ShapeDtypeStruct((B, 6 * H), f32),
            jax.ShapeDtypeStruct((1, H), f32),
        ],
        scratch_shapes=[
            pltpu.VMEM((T, NB, 3 * H), f32),
            pltpu.VMEM((B, 6 * H), f32),
        ],
    )(xs, w0, b0, wa, wc, bc, wih, whh, bih, bhh, e1, e2,
      ref_embs, dp)

    return out, ltot[0, 0]
